# baseline (speedup 1.0000x reference)
import functools

import jax
import jax.numpy as jnp
from jax import lax
from jax.experimental import pallas as pl
from jax.experimental.pallas import tpu as pltpu
from jax.experimental.pallas import tpu_sc as plsc

N = 10000
E = 320000
D = 128

NC = 2
NS = 16
LN = 16

K = 64
CH = 156
KT = 16
EPW = CH * K + KT
EA = NC * NS * CH * K
IDXB = 16
NBLK = CH // IDXB
RPW = 632
RPW_LAST = N - 15 * RPW

_I32 = jnp.int32
_HI_MASK = -65536
_RND = 0x8000

_MESH = plsc.VectorSubcoreMesh(
    core_axis_name="c", subcore_axis_name="s", num_cores=NC, num_subcores=NS
)


def _sc_body(has_t, write_e, *refs):
    it = iter(refs)
    sd3 = next(it)
    tidx = next(it)
    p1 = next(it)
    p2 = next(it)
    t = next(it) if has_t else None
    e_out = next(it) if write_e else None
    agg_a = next(it)
    agg_b = next(it)
    idxs = next(it)
    idxt = next(it)
    abufs = (next(it), next(it))
    bbufs = (next(it), next(it))
    tbufs = (next(it), next(it))
    ebuf = next(it)
    agg_sh = next(it)
    sems_a = (next(it), next(it))
    sems_b = (next(it), next(it))
    sems_t = (next(it), next(it))

    c = lax.axis_index("c")
    s = lax.axis_index("s")
    wid = c * NS + s

    z0 = abufs[0]

    def _zrow(r, _):
        for j in range(D // LN):
            z0[r, pl.ds(j * LN, LN)] = jnp.zeros((LN,), jnp.float32)
        return 0

    lax.fori_loop(0, K, _zrow, 0)

    def _zero_span(base, rows):
        for q in range(rows // K):
            pltpu.sync_copy(z0, agg_sh.at[pl.ds(base + q * K, K)])
        rem = rows % K
        if rem:
            pltpu.sync_copy(z0.at[pl.ds(0, rem)],
                            agg_sh.at[pl.ds(base + rows - rem, rem)])

    @pl.when(s < NS - 1)
    def _():
        _zero_span(s * RPW, RPW)

    @pl.when(s == NS - 1)
    def _():
        _zero_span((NS - 1) * RPW, RPW_LAST)

    plsc.subcore_barrier()

    def _issue(cl, base_c, slot):
        prow = (wid * CH + base_c + cl) * (K // 2)
        pltpu.async_copy(p1.at[idxs.at[2 * cl]], abufs[slot], sems_a[slot])
        pltpu.async_copy(p2.at[idxs.at[2 * cl + 1]], bbufs[slot],
                         sems_b[slot])
        if has_t:
            pltpu.async_copy(t.at[pl.ds(prow, K // 2)], tbufs[slot],
                             sems_t[slot])

    def _wait(slot):
        dummy = p1.at[pl.ds(0, K)]
        pltpu.make_async_copy(dummy, abufs[slot], sems_a[slot]).wait()
        pltpu.make_async_copy(dummy, bbufs[slot], sems_b[slot]).wait()
        if has_t:
            dummy_t = t.at[pl.ds(0, K // 2)]
            pltpu.make_async_copy(dummy_t, tbufs[slot], sems_t[slot]).wait()

    def _pair_rows(ab, bb, tb, nrows):
        @plsc.parallel_loop(0, nrows, unroll=2)
        def _row(rr):
            r0 = 2 * rr
            r1 = r0 + 1
            for g in range(D // LN):
                sl = pl.ds(g * LN, LN)
                u = ab[r0, sl] + bb[r0, sl]
                v = ab[r1, sl] + bb[r1, sl]
                if has_t:
                    tw = tb[rr, sl]
                    u = u + lax.bitcast_convert_type(tw << 16, jnp.float32)
                    v = v + lax.bitcast_convert_type(tw & _HI_MASK,
                                                     jnp.float32)
                u = jnp.maximum(u, 0.0)
                v = jnp.maximum(v, 0.0)
                ab[r0, sl] = u
                ab[r1, sl] = v
                if write_e:
                    ub = lax.bitcast_convert_type(u, _I32)
                    vb = lax.bitcast_convert_type(v, _I32)
                    ebuf[rr, sl] = (
                        lax.shift_right_logical(ub + _RND, 16)
                        | ((vb + _RND) & _HI_MASK)
                    )

    def _run_block(base_c, bsz):
        pltpu.sync_copy(sd3.at[wid].at[pl.ds(2 * base_c, 2 * bsz)],
                        idxs.at[pl.ds(0, 2 * bsz)])
        _issue(0, base_c, 0)
        _issue(1, base_c, 1)

        def _pair(j2, _):
            for slot in (0, 1):
                cl = j2 * 2 + slot
                prow = (wid * CH + base_c + cl) * (K // 2)
                _wait(slot)
                _pair_rows(abufs[slot], bbufs[slot], tbufs[slot], K // 2)
                pltpu.sync_copy(abufs[slot], agg_sh.at[idxs.at[2 * cl + 1]],
                                add=True)
                if write_e:
                    pltpu.sync_copy(ebuf, e_out.at[pl.ds(prow, K // 2)])

                @pl.when(cl + 2 < bsz)
                def _(cl=cl, slot=slot):
                    _issue(cl + 2, base_c, slot)
            return 0

        lax.fori_loop(0, bsz // 2, _pair, 0)

    def _block(blk, _):
        _run_block(pl.multiple_of(blk * IDXB, IDXB), IDXB)
        return 0

    lax.fori_loop(0, NBLK, _block, 0)
    if CH % IDXB:
        _run_block(NBLK * IDXB, CH % IDXB)

    pltpu.sync_copy(tidx.at[wid], idxt)
    tprow = EA // 2 + wid * (KT // 2)
    cpa = pltpu.async_copy(p1.at[idxt.at[0]], abufs[0].at[pl.ds(0, KT)],
                           sems_a[0])
    cpb = pltpu.async_copy(p2.at[idxt.at[1]], bbufs[0].at[pl.ds(0, KT)],
                           sems_b[0])
    if has_t:
        cpt = pltpu.async_copy(t.at[pl.ds(tprow, KT // 2)],
                               tbufs[0].at[pl.ds(0, KT // 2)], sems_t[0])
    cpa.wait()
    cpb.wait()
    if has_t:
        cpt.wait()
    _pair_rows(abufs[0], bbufs[0], tbufs[0], KT // 2)
    pltpu.sync_copy(abufs[0].at[pl.ds(0, KT)], agg_sh.at[idxt.at[1]],
                    add=True)
    if write_e:
        pltpu.sync_copy(ebuf.at[pl.ds(0, KT // 2)],
                        e_out.at[pl.ds(tprow, KT // 2)])

    plsc.subcore_barrier()

    def _flush(agg_out):
        @pl.when(s < NS - 1)
        def _():
            pltpu.sync_copy(agg_sh.at[pl.ds(s * RPW, RPW)],
                            agg_out.at[pl.ds(s * RPW, RPW)])

        @pl.when(s == NS - 1)
        def _():
            pltpu.sync_copy(agg_sh.at[pl.ds((NS - 1) * RPW, RPW_LAST)],
                            agg_out.at[pl.ds((NS - 1) * RPW, RPW_LAST)])

    @pl.when(c == 0)
    def _():
        _flush(agg_a)

    @pl.when(c == 1)
    def _():
        _flush(agg_b)


def _make_sc_kernel(has_t, write_e):
    f32 = jnp.float32
    outs = []
    if write_e:
        outs.append(jax.ShapeDtypeStruct((E // 2, D), jnp.int32))
    outs.append(jax.ShapeDtypeStruct((N, D), f32))
    outs.append(jax.ShapeDtypeStruct((N, D), f32))
    scratch = (
        [pltpu.VMEM((2 * IDXB, K), jnp.int32)]
        + [pltpu.VMEM((2, KT), jnp.int32)]
        + [pltpu.VMEM((K, D), f32)] * 4
        + [pltpu.VMEM((K // 2, D), jnp.int32)] * 2
        + [pltpu.VMEM((K // 2, D), jnp.int32)]
        + [pltpu.VMEM_SHARED((N, D), f32)]
        + [pltpu.SemaphoreType.DMA] * 6
    )
    return pl.kernel(
        functools.partial(_sc_body, has_t, write_e),
        out_type=tuple(outs),
        mesh=_MESH,
        scratch_types=scratch,
    )


_BMN = 2000
_BME = 2000


def _wspec():
    return pl.BlockSpec((D, D), lambda i: (0, 0))


def _bspec():
    return pl.BlockSpec((1, D), lambda i: (0, 0))


def _rows(bm):
    return pl.BlockSpec((bm, D), lambda i: (i, 0))


def _prep0_body(x_ref, w1_ref, w2_ref, be_ref, p1_ref, p2_ref):
    x = x_ref[...]
    p1_ref[...] = jnp.dot(x, w1_ref[...], preferred_element_type=jnp.float32)
    p2_ref[...] = (
        jnp.dot(x, w2_ref[...], preferred_element_type=jnp.float32)
        + be_ref[...]
    )


def _prep0(z, w1, w2, be):
    return pl.pallas_call(
        _prep0_body,
        grid=(N // _BMN,),
        in_specs=[_rows(_BMN), _wspec(), _wspec(), _bspec()],
        out_specs=[_rows(_BMN), _rows(_BMN)],
        out_shape=[jax.ShapeDtypeStruct((N, D), jnp.float32)] * 2,
    )(z, w1, w2, be)


def _edge_mm_body(e_ref, w_ref, o_ref):
    eb = pltpu.bitcast(e_ref[...], jnp.bfloat16)
    tt = jnp.dot(eb, w_ref[...], preferred_element_type=jnp.float32)
    o_ref[...] = pltpu.bitcast(tt.astype(jnp.bfloat16), jnp.int32)


def _edge_mm(e0, w3):
    return pl.pallas_call(
        _edge_mm_body,
        grid=(E // 2 // _BME,),
        in_specs=[_rows(_BME), _wspec()],
        out_specs=_rows(_BME),
        out_shape=jax.ShapeDtypeStruct((E // 2, D), jnp.int32),
    )(e0, w3)


def _edge_mm2_body(e0_ref, e1_ref, w_ref, o_ref):
    ea = pltpu.bitcast(e0_ref[...], jnp.bfloat16).astype(jnp.float32)
    eb = pltpu.bitcast(e1_ref[...], jnp.bfloat16).astype(jnp.float32)
    tt = jnp.dot((ea + eb).astype(jnp.bfloat16), w_ref[...],
                 preferred_element_type=jnp.float32)
    o_ref[...] = pltpu.bitcast(tt.astype(jnp.bfloat16), jnp.int32)


def _edge_mm2(e0, e1, w3):
    return pl.pallas_call(
        _edge_mm2_body,
        grid=(E // 2 // _BME,),
        in_specs=[_rows(_BME), _rows(_BME), _wspec()],
        out_specs=_rows(_BME),
        out_shape=jax.ShapeDtypeStruct((E // 2, D), jnp.int32),
    )(e0, e1, w3)


def _node_body(residual, prep, *refs):
    it = iter(refs)
    x_ref = next(it)
    aa_ref = next(it)
    ab_ref = next(it)
    wna_ref = next(it)
    wnb_ref = next(it)
    bn_ref = next(it)
    if prep:
        w1_ref = next(it)
        w2_ref = next(it)
        be_ref = next(it)
    h_ref = next(it)
    if prep:
        p1_ref = next(it)
        p2_ref = next(it)
    x = x_ref[...]
    agg = aa_ref[...] + ab_ref[...]
    h = jnp.maximum(
        jnp.dot(x, wna_ref[...], preferred_element_type=jnp.float32)
        + jnp.dot(agg, wnb_ref[...], preferred_element_type=jnp.float32)
        + bn_ref[...],
        0.0,
    )
    if residual:
        h = h + x
    h_ref[...] = h
    if prep:
        p1_ref[...] = jnp.dot(
            h, w1_ref[...], preferred_element_type=jnp.float32
        )
        p2_ref[...] = (
            jnp.dot(h, w2_ref[...], preferred_element_type=jnp.float32)
            + be_ref[...]
        )


def _node(residual, prep, x, agg_a, agg_b, wna, wnb, bn, *prep_args):
    n_out = 3 if prep else 1
    in_specs = [_rows(_BMN)] * 3 + [_wspec(), _wspec(), _bspec()]
    if prep:
        in_specs += [_wspec(), _wspec(), _bspec()]
    res = pl.pallas_call(
        functools.partial(_node_body, residual, prep),
        grid=(N // _BMN,),
        in_specs=in_specs,
        out_specs=[_rows(_BMN)] * n_out,
        out_shape=[jax.ShapeDtypeStruct((N, D), jnp.float32)] * n_out,
    )(x, agg_a, agg_b, wna, wnb, bn, *prep_args)
    return res if prep else res[0]


def kernel(edge_index, z, We0, be0, Wn0, bn0, We1, be1, Wn1, bn1,
           We2, be2, Wn2, bn2):
    nw = NC * NS
    src = edge_index[0]
    dst = edge_index[1]
    sd3 = jnp.transpose(
        edge_index[:, :EA].reshape(2, nw, CH, K), (1, 2, 0, 3)
    ).reshape(nw, 2 * CH, K)
    tidx = jnp.transpose(edge_index[:, EA:].reshape(2, nw, KT), (1, 0, 2))

    be0r = be0.reshape(1, D)
    be1r = be1.reshape(1, D)
    be2r = be2.reshape(1, D)
    bn0r = bn0.reshape(1, D)
    bn1r = bn1.reshape(1, D)
    bn2r = bn2.reshape(1, D)

    sc0 = _make_sc_kernel(has_t=False, write_e=True)
    sc1 = _make_sc_kernel(has_t=True, write_e=True)
    sc2 = _make_sc_kernel(has_t=True, write_e=False)

    p1, p2 = _prep0(z, We0[:D], We0[D:], be0r)
    e0, agg_a, agg_b = sc0(sd3, tidx, p1, p2)
    x1, p1, p2 = _node(False, True, z, agg_a, agg_b,
                       Wn0[:D], Wn0[D:], bn0r, We1[:D], We1[D:2 * D], be1r)

    t1 = _edge_mm(e0, We1[2 * D:].astype(jnp.bfloat16))
    e1, agg_a, agg_b = sc1(sd3, tidx, p1, p2, t1)
    x2, p1, p2 = _node(True, True, x1, agg_a, agg_b,
                       Wn1[:D], Wn1[D:], bn1r, We2[:D], We2[D:2 * D], be2r)

    t2 = _edge_mm2(e0, e1, We2[2 * D:].astype(jnp.bfloat16))
    agg_a, agg_b = sc2(sd3, tidx, p1, p2, t2)
    out = _node(False, False, x2, agg_a, agg_b, Wn2[:D], Wn2[D:], bn2r)
    return out


# speedup vs baseline: 1.0483x; 1.0483x over previous
import functools

import jax
import jax.numpy as jnp
from jax import lax
from jax.experimental import pallas as pl
from jax.experimental.pallas import tpu as pltpu
from jax.experimental.pallas import tpu_sc as plsc

N = 10000
E = 320000
D = 128

NC = 2
NS = 16
LN = 16

K = 64
CH = 156
KT = 16
EPW = CH * K + KT
EA = NC * NS * CH * K
IDXB = 16
NBLK = CH // IDXB
RPW = 632
RPW_LAST = N - 15 * RPW

_I32 = jnp.int32
_HI_MASK = -65536
_RND = 0x8000

_MESH = plsc.VectorSubcoreMesh(
    core_axis_name="c", subcore_axis_name="s", num_cores=NC, num_subcores=NS
)


def _sc_body(has_t, write_e, *refs):
    it = iter(refs)
    sd3 = next(it)
    tidx = next(it)
    p1 = next(it)
    p2 = next(it)
    t = next(it) if has_t else None
    e_out = next(it) if write_e else None
    agg_a = next(it)
    agg_b = next(it)
    idxs = next(it)
    idxt = next(it)
    abufs = (next(it), next(it))
    bbufs = (next(it), next(it))
    tbufs = (next(it), next(it))
    ebuf = next(it)
    agg_sh = next(it)
    sems_a = (next(it), next(it))
    sems_b = (next(it), next(it))
    sems_t = (next(it), next(it))

    c = lax.axis_index("c")
    s = lax.axis_index("s")
    wid = c * NS + s

    z0 = abufs[0]

    def _zrow(r, _):
        for j in range(D // LN):
            z0[r, pl.ds(j * LN, LN)] = jnp.zeros((LN,), jnp.float32)
        return 0

    lax.fori_loop(0, K, _zrow, 0)

    def _zero_span(base, rows):
        for q in range(rows // K):
            pltpu.sync_copy(z0, agg_sh.at[pl.ds(base + q * K, K)])
        rem = rows % K
        if rem:
            pltpu.sync_copy(z0.at[pl.ds(0, rem)],
                            agg_sh.at[pl.ds(base + rows - rem, rem)])

    @pl.when(s < NS - 1)
    def _():
        _zero_span(s * RPW, RPW)

    @pl.when(s == NS - 1)
    def _():
        _zero_span((NS - 1) * RPW, RPW_LAST)

    plsc.subcore_barrier()

    def _issue(cl, base_c, slot):
        prow = (wid * CH + base_c + cl) * (K // 2)
        pltpu.async_copy(p1.at[idxs.at[2 * cl]], abufs[slot], sems_a[slot])
        pltpu.async_copy(p2.at[idxs.at[2 * cl + 1]], bbufs[slot],
                         sems_b[slot])
        if has_t:
            pltpu.async_copy(t.at[pl.ds(prow, K // 2)], tbufs[slot],
                             sems_t[slot])

    def _wait(slot):
        dummy = p1.at[pl.ds(0, K)]
        pltpu.make_async_copy(dummy, abufs[slot], sems_a[slot]).wait()
        pltpu.make_async_copy(dummy, bbufs[slot], sems_b[slot]).wait()
        if has_t:
            dummy_t = t.at[pl.ds(0, K // 2)]
            pltpu.make_async_copy(dummy_t, tbufs[slot], sems_t[slot]).wait()

    def _pair_rows(ab, bb, tb, nrows):
        @plsc.parallel_loop(0, nrows, unroll=2)
        def _row(rr):
            r0 = 2 * rr
            r1 = r0 + 1
            for g in range(D // LN):
                sl = pl.ds(g * LN, LN)
                u = ab[r0, sl] + bb[r0, sl]
                v = ab[r1, sl] + bb[r1, sl]
                if has_t:
                    tw = tb[rr, sl]
                    u = u + lax.bitcast_convert_type(tw << 16, jnp.float32)
                    v = v + lax.bitcast_convert_type(tw & _HI_MASK,
                                                     jnp.float32)
                u = jnp.maximum(u, 0.0)
                v = jnp.maximum(v, 0.0)
                ab[r0, sl] = u
                ab[r1, sl] = v
                if write_e:
                    ub = lax.bitcast_convert_type(u, _I32)
                    vb = lax.bitcast_convert_type(v, _I32)
                    ebuf[rr, sl] = (
                        lax.shift_right_logical(ub + _RND, 16)
                        | ((vb + _RND) & _HI_MASK)
                    )

    def _run_block(base_c, bsz):
        pltpu.sync_copy(sd3.at[wid].at[pl.ds(2 * base_c, 2 * bsz)],
                        idxs.at[pl.ds(0, 2 * bsz)])
        _issue(0, base_c, 0)
        _issue(1, base_c, 1)

        def _pair(j2, _):
            for slot in (0, 1):
                cl = j2 * 2 + slot
                prow = (wid * CH + base_c + cl) * (K // 2)
                _wait(slot)
                _pair_rows(abufs[slot], bbufs[slot], tbufs[slot], K // 2)
                pltpu.sync_copy(abufs[slot], agg_sh.at[idxs.at[2 * cl + 1]],
                                add=True)
                if write_e:
                    pltpu.sync_copy(ebuf, e_out.at[pl.ds(prow, K // 2)])

                @pl.when(cl + 2 < bsz)
                def _(cl=cl, slot=slot):
                    _issue(cl + 2, base_c, slot)
            return 0

        lax.fori_loop(0, bsz // 2, _pair, 0)

    def _block(blk, _):
        _run_block(pl.multiple_of(blk * IDXB, IDXB), IDXB)
        return 0

    lax.fori_loop(0, NBLK, _block, 0)
    if CH % IDXB:
        _run_block(NBLK * IDXB, CH % IDXB)

    pltpu.sync_copy(tidx.at[wid], idxt)
    tprow = EA // 2 + wid * (KT // 2)
    cpa = pltpu.async_copy(p1.at[idxt.at[0]], abufs[0].at[pl.ds(0, KT)],
                           sems_a[0])
    cpb = pltpu.async_copy(p2.at[idxt.at[1]], bbufs[0].at[pl.ds(0, KT)],
                           sems_b[0])
    if has_t:
        cpt = pltpu.async_copy(t.at[pl.ds(tprow, KT // 2)],
                               tbufs[0].at[pl.ds(0, KT // 2)], sems_t[0])
    cpa.wait()
    cpb.wait()
    if has_t:
        cpt.wait()
    _pair_rows(abufs[0], bbufs[0], tbufs[0], KT // 2)
    pltpu.sync_copy(abufs[0].at[pl.ds(0, KT)], agg_sh.at[idxt.at[1]],
                    add=True)
    if write_e:
        pltpu.sync_copy(ebuf.at[pl.ds(0, KT // 2)],
                        e_out.at[pl.ds(tprow, KT // 2)])

    plsc.subcore_barrier()

    def _flush(agg_out):
        @pl.when(s < NS - 1)
        def _():
            pltpu.sync_copy(agg_sh.at[pl.ds(s * RPW, RPW)],
                            agg_out.at[pl.ds(s * RPW, RPW)])

        @pl.when(s == NS - 1)
        def _():
            pltpu.sync_copy(agg_sh.at[pl.ds((NS - 1) * RPW, RPW_LAST)],
                            agg_out.at[pl.ds((NS - 1) * RPW, RPW_LAST)])

    @pl.when(c == 0)
    def _():
        _flush(agg_a)

    @pl.when(c == 1)
    def _():
        _flush(agg_b)


def _make_sc_kernel(has_t, write_e):
    f32 = jnp.float32
    outs = []
    if write_e:
        outs.append(jax.ShapeDtypeStruct((E // 2, D), jnp.int32))
    outs.append(jax.ShapeDtypeStruct((N, D), f32))
    outs.append(jax.ShapeDtypeStruct((N, D), f32))
    scratch = (
        [pltpu.VMEM((2 * IDXB, K), jnp.int32)]
        + [pltpu.VMEM((2, KT), jnp.int32)]
        + [pltpu.VMEM((K, D), f32)] * 4
        + [pltpu.VMEM((K // 2, D), jnp.int32)] * 2
        + [pltpu.VMEM((K // 2, D), jnp.int32)]
        + [pltpu.VMEM_SHARED((N, D), f32)]
        + [pltpu.SemaphoreType.DMA] * 6
    )
    return pl.kernel(
        functools.partial(_sc_body, has_t, write_e),
        out_type=tuple(outs),
        mesh=_MESH,
        scratch_types=scratch,
    )


_BMN = 2000
_BME = 4000


def _wspec():
    return pl.BlockSpec((D, D), lambda i: (0, 0))


def _bspec():
    return pl.BlockSpec((1, D), lambda i: (0, 0))


def _rows(bm):
    return pl.BlockSpec((bm, D), lambda i: (i, 0))


def _prep0_body(x_ref, w1_ref, w2_ref, be_ref, p1_ref, p2_ref):
    x = x_ref[...]
    p1_ref[...] = jnp.dot(x, w1_ref[...], preferred_element_type=jnp.float32)
    p2_ref[...] = (
        jnp.dot(x, w2_ref[...], preferred_element_type=jnp.float32)
        + be_ref[...]
    )


def _prep0(z, w1, w2, be):
    return pl.pallas_call(
        _prep0_body,
        grid=(N // _BMN,),
        in_specs=[_rows(_BMN), _wspec(), _wspec(), _bspec()],
        out_specs=[_rows(_BMN), _rows(_BMN)],
        out_shape=[jax.ShapeDtypeStruct((N, D), jnp.float32)] * 2,
    )(z, w1, w2, be)


def _edge_mm_body(e_ref, w_ref, o_ref):
    eb = pltpu.bitcast(e_ref[...], jnp.bfloat16)
    tt = jnp.dot(eb, w_ref[...], preferred_element_type=jnp.float32)
    o_ref[...] = pltpu.bitcast(tt.astype(jnp.bfloat16), jnp.int32)


def _edge_mm(e0, w3):
    return pl.pallas_call(
        _edge_mm_body,
        grid=(E // 2 // _BME,),
        in_specs=[_rows(_BME), _wspec()],
        out_specs=_rows(_BME),
        out_shape=jax.ShapeDtypeStruct((E // 2, D), jnp.int32),
    )(e0, w3)


def _edge_mm2_body(e0_ref, e1_ref, w_ref, o_ref):
    ea = pltpu.bitcast(e0_ref[...], jnp.bfloat16).astype(jnp.float32)
    eb = pltpu.bitcast(e1_ref[...], jnp.bfloat16).astype(jnp.float32)
    tt = jnp.dot((ea + eb).astype(jnp.bfloat16), w_ref[...],
                 preferred_element_type=jnp.float32)
    o_ref[...] = pltpu.bitcast(tt.astype(jnp.bfloat16), jnp.int32)


def _edge_mm2(e0, e1, w3):
    return pl.pallas_call(
        _edge_mm2_body,
        grid=(E // 2 // _BME,),
        in_specs=[_rows(_BME), _rows(_BME), _wspec()],
        out_specs=_rows(_BME),
        out_shape=jax.ShapeDtypeStruct((E // 2, D), jnp.int32),
    )(e0, e1, w3)


def _node_body(residual, prep, *refs):
    it = iter(refs)
    x_ref = next(it)
    aa_ref = next(it)
    ab_ref = next(it)
    wna_ref = next(it)
    wnb_ref = next(it)
    bn_ref = next(it)
    if prep:
        w1_ref = next(it)
        w2_ref = next(it)
        be_ref = next(it)
    h_ref = next(it)
    if prep:
        p1_ref = next(it)
        p2_ref = next(it)
    x = x_ref[...]
    agg = aa_ref[...] + ab_ref[...]
    h = jnp.maximum(
        jnp.dot(x, wna_ref[...], preferred_element_type=jnp.float32)
        + jnp.dot(agg, wnb_ref[...], preferred_element_type=jnp.float32)
        + bn_ref[...],
        0.0,
    )
    if residual:
        h = h + x
    h_ref[...] = h
    if prep:
        p1_ref[...] = jnp.dot(
            h, w1_ref[...], preferred_element_type=jnp.float32
        )
        p2_ref[...] = (
            jnp.dot(h, w2_ref[...], preferred_element_type=jnp.float32)
            + be_ref[...]
        )


def _node(residual, prep, x, agg_a, agg_b, wna, wnb, bn, *prep_args):
    n_out = 3 if prep else 1
    in_specs = [_rows(_BMN)] * 3 + [_wspec(), _wspec(), _bspec()]
    if prep:
        in_specs += [_wspec(), _wspec(), _bspec()]
    res = pl.pallas_call(
        functools.partial(_node_body, residual, prep),
        grid=(N // _BMN,),
        in_specs=in_specs,
        out_specs=[_rows(_BMN)] * n_out,
        out_shape=[jax.ShapeDtypeStruct((N, D), jnp.float32)] * n_out,
    )(x, agg_a, agg_b, wna, wnb, bn, *prep_args)
    return res if prep else res[0]


def kernel(edge_index, z, We0, be0, Wn0, bn0, We1, be1, Wn1, bn1,
           We2, be2, Wn2, bn2):
    nw = NC * NS
    src = edge_index[0]
    dst = edge_index[1]
    sd3 = jnp.transpose(
        edge_index[:, :EA].reshape(2, nw, CH, K), (1, 2, 0, 3)
    ).reshape(nw, 2 * CH, K)
    tidx = jnp.transpose(edge_index[:, EA:].reshape(2, nw, KT), (1, 0, 2))

    be0r = be0.reshape(1, D)
    be1r = be1.reshape(1, D)
    be2r = be2.reshape(1, D)
    bn0r = bn0.reshape(1, D)
    bn1r = bn1.reshape(1, D)
    bn2r = bn2.reshape(1, D)

    sc0 = _make_sc_kernel(has_t=False, write_e=True)
    sc1 = _make_sc_kernel(has_t=True, write_e=True)
    sc2 = _make_sc_kernel(has_t=True, write_e=False)

    p1, p2 = _prep0(z, We0[:D], We0[D:], be0r)
    e0, agg_a, agg_b = sc0(sd3, tidx, p1, p2)
    x1, p1, p2 = _node(False, True, z, agg_a, agg_b,
                       Wn0[:D], Wn0[D:], bn0r, We1[:D], We1[D:2 * D], be1r)

    t1 = _edge_mm(e0, We1[2 * D:].astype(jnp.bfloat16))
    e1, agg_a, agg_b = sc1(sd3, tidx, p1, p2, t1)
    x2, p1, p2 = _node(True, True, x1, agg_a, agg_b,
                       Wn1[:D], Wn1[D:], bn1r, We2[:D], We2[D:2 * D], be2r)

    t2 = _edge_mm2(e0, e1, We2[2 * D:].astype(jnp.bfloat16))
    agg_a, agg_b = sc2(sd3, tidx, p1, p2, t2)
    out = _node(False, False, x2, agg_a, agg_b, Wn2[:D], Wn2[D:], bn2r)
    return out


# speedup vs baseline: 1.0606x; 1.0118x over previous
import functools

import jax
import jax.numpy as jnp
from jax import lax
from jax.experimental import pallas as pl
from jax.experimental.pallas import tpu as pltpu
from jax.experimental.pallas import tpu_sc as plsc

N = 10000
E = 320000
D = 128

NC = 2
NS = 16
LN = 16

K = 64
CH = 156
KT = 16
EPW = CH * K + KT
EA = NC * NS * CH * K
IDXB = 16
NBLK = CH // IDXB
RPW = 632
RPW_LAST = N - 15 * RPW

_I32 = jnp.int32
_HI_MASK = -65536
_RND = 0x8000

_MESH = plsc.VectorSubcoreMesh(
    core_axis_name="c", subcore_axis_name="s", num_cores=NC, num_subcores=NS
)


def _sc_body(has_t, write_e, *refs):
    it = iter(refs)
    sd3 = next(it)
    tidx = next(it)
    p1 = next(it)
    p2 = next(it)
    t = next(it) if has_t else None
    e_out = next(it) if write_e else None
    agg_a = next(it)
    agg_b = next(it)
    idxs = next(it)
    idxt = next(it)
    abufs = (next(it), next(it))
    bbufs = (next(it), next(it))
    tbufs = (next(it), next(it))
    ebuf = next(it)
    agg_sh = next(it)
    sems_a = (next(it), next(it))
    sems_b = (next(it), next(it))
    sems_t = (next(it), next(it))

    c = lax.axis_index("c")
    s = lax.axis_index("s")
    wid = c * NS + s

    z0 = abufs[0]

    def _zrow(r, _):
        for j in range(D // LN):
            z0[r, pl.ds(j * LN, LN)] = jnp.zeros((LN,), jnp.float32)
        return 0

    lax.fori_loop(0, K, _zrow, 0)

    def _zero_span(base, rows):
        for q in range(rows // K):
            pltpu.sync_copy(z0, agg_sh.at[pl.ds(base + q * K, K)])
        rem = rows % K
        if rem:
            pltpu.sync_copy(z0.at[pl.ds(0, rem)],
                            agg_sh.at[pl.ds(base + rows - rem, rem)])

    @pl.when(s < NS - 1)
    def _():
        _zero_span(s * RPW, RPW)

    @pl.when(s == NS - 1)
    def _():
        _zero_span((NS - 1) * RPW, RPW_LAST)

    plsc.subcore_barrier()

    def _issue(cl, base_c, slot):
        prow = (wid * CH + base_c + cl) * (K // 2)
        pltpu.async_copy(p1.at[idxs.at[2 * cl]], abufs[slot], sems_a[slot])
        pltpu.async_copy(p2.at[idxs.at[2 * cl + 1]], bbufs[slot],
                         sems_b[slot])
        if has_t:
            pltpu.async_copy(t.at[pl.ds(prow, K // 2)], tbufs[slot],
                             sems_t[slot])

    def _wait(slot):
        dummy = p1.at[pl.ds(0, K)]
        pltpu.make_async_copy(dummy, abufs[slot], sems_a[slot]).wait()
        pltpu.make_async_copy(dummy, bbufs[slot], sems_b[slot]).wait()
        if has_t:
            dummy_t = t.at[pl.ds(0, K // 2)]
            pltpu.make_async_copy(dummy_t, tbufs[slot], sems_t[slot]).wait()

    def _pair_rows(ab, bb, tb, nrows):
        @plsc.parallel_loop(0, nrows, unroll=2)
        def _row(rr):
            r0 = 2 * rr
            r1 = r0 + 1
            for g in range(D // LN):
                sl = pl.ds(g * LN, LN)
                u = ab[r0, sl] + bb[r0, sl]
                v = ab[r1, sl] + bb[r1, sl]
                if has_t:
                    tw = tb[rr, sl]
                    u = u + lax.bitcast_convert_type(tw << 16, jnp.float32)
                    v = v + lax.bitcast_convert_type(tw & _HI_MASK,
                                                     jnp.float32)
                u = jnp.maximum(u, 0.0)
                v = jnp.maximum(v, 0.0)
                ab[r0, sl] = u
                ab[r1, sl] = v
                if write_e:
                    ub = lax.bitcast_convert_type(u, _I32)
                    vb = lax.bitcast_convert_type(v, _I32)
                    ebuf[rr, sl] = (
                        lax.shift_right_logical(ub + _RND, 16)
                        | ((vb + _RND) & _HI_MASK)
                    )

    def _run_block(base_c, bsz):
        pltpu.sync_copy(sd3.at[wid].at[pl.ds(2 * base_c, 2 * bsz)],
                        idxs.at[pl.ds(0, 2 * bsz)])
        _issue(0, base_c, 0)
        _issue(1, base_c, 1)

        def _pair(j2, _):
            for slot in (0, 1):
                cl = j2 * 2 + slot
                prow = (wid * CH + base_c + cl) * (K // 2)
                _wait(slot)
                _pair_rows(abufs[slot], bbufs[slot], tbufs[slot], K // 2)
                pltpu.sync_copy(abufs[slot], agg_sh.at[idxs.at[2 * cl + 1]],
                                add=True)
                if write_e:
                    pltpu.sync_copy(ebuf, e_out.at[pl.ds(prow, K // 2)])

                @pl.when(cl + 2 < bsz)
                def _(cl=cl, slot=slot):
                    _issue(cl + 2, base_c, slot)
            return 0

        lax.fori_loop(0, bsz // 2, _pair, 0)

    def _block(blk, _):
        _run_block(pl.multiple_of(blk * IDXB, IDXB), IDXB)
        return 0

    lax.fori_loop(0, NBLK, _block, 0)
    if CH % IDXB:
        _run_block(NBLK * IDXB, CH % IDXB)

    pltpu.sync_copy(tidx.at[wid], idxt)
    tprow = EA // 2 + wid * (KT // 2)
    cpa = pltpu.async_copy(p1.at[idxt.at[0]], abufs[0].at[pl.ds(0, KT)],
                           sems_a[0])
    cpb = pltpu.async_copy(p2.at[idxt.at[1]], bbufs[0].at[pl.ds(0, KT)],
                           sems_b[0])
    if has_t:
        cpt = pltpu.async_copy(t.at[pl.ds(tprow, KT // 2)],
                               tbufs[0].at[pl.ds(0, KT // 2)], sems_t[0])
    cpa.wait()
    cpb.wait()
    if has_t:
        cpt.wait()
    _pair_rows(abufs[0], bbufs[0], tbufs[0], KT // 2)
    pltpu.sync_copy(abufs[0].at[pl.ds(0, KT)], agg_sh.at[idxt.at[1]],
                    add=True)
    if write_e:
        pltpu.sync_copy(ebuf.at[pl.ds(0, KT // 2)],
                        e_out.at[pl.ds(tprow, KT // 2)])

    plsc.subcore_barrier()

    def _flush(agg_out):
        @pl.when(s < NS - 1)
        def _():
            pltpu.sync_copy(agg_sh.at[pl.ds(s * RPW, RPW)],
                            agg_out.at[pl.ds(s * RPW, RPW)])

        @pl.when(s == NS - 1)
        def _():
            pltpu.sync_copy(agg_sh.at[pl.ds((NS - 1) * RPW, RPW_LAST)],
                            agg_out.at[pl.ds((NS - 1) * RPW, RPW_LAST)])

    @pl.when(c == 0)
    def _():
        _flush(agg_a)

    @pl.when(c == 1)
    def _():
        _flush(agg_b)


def _make_sc_kernel(has_t, write_e):
    f32 = jnp.float32
    outs = []
    if write_e:
        outs.append(jax.ShapeDtypeStruct((E // 2, D), jnp.int32))
    outs.append(jax.ShapeDtypeStruct((N, D), f32))
    outs.append(jax.ShapeDtypeStruct((N, D), f32))
    scratch = (
        [pltpu.VMEM((2 * IDXB, K), jnp.int32)]
        + [pltpu.VMEM((2, KT), jnp.int32)]
        + [pltpu.VMEM((K, D), f32)] * 4
        + [pltpu.VMEM((K // 2, D), jnp.int32)] * 2
        + [pltpu.VMEM((K // 2, D), jnp.int32)]
        + [pltpu.VMEM_SHARED((N, D), f32)]
        + [pltpu.SemaphoreType.DMA] * 6
    )
    return pl.kernel(
        functools.partial(_sc_body, has_t, write_e),
        out_type=tuple(outs),
        mesh=_MESH,
        scratch_types=scratch,
    )


_BMN = 2000
_BME = 8000


def _wspec():
    return pl.BlockSpec((D, D), lambda i: (0, 0))


def _bspec():
    return pl.BlockSpec((1, D), lambda i: (0, 0))


def _rows(bm):
    return pl.BlockSpec((bm, D), lambda i: (i, 0))


def _prep0_body(x_ref, w1_ref, w2_ref, be_ref, p1_ref, p2_ref):
    x = x_ref[...]
    p1_ref[...] = jnp.dot(x, w1_ref[...], preferred_element_type=jnp.float32)
    p2_ref[...] = (
        jnp.dot(x, w2_ref[...], preferred_element_type=jnp.float32)
        + be_ref[...]
    )


def _prep0(z, w1, w2, be):
    return pl.pallas_call(
        _prep0_body,
        grid=(N // _BMN,),
        in_specs=[_rows(_BMN), _wspec(), _wspec(), _bspec()],
        out_specs=[_rows(_BMN), _rows(_BMN)],
        out_shape=[jax.ShapeDtypeStruct((N, D), jnp.float32)] * 2,
    )(z, w1, w2, be)


def _edge_mm_body(e_ref, w_ref, o_ref):
    eb = pltpu.bitcast(e_ref[...], jnp.bfloat16)
    tt = jnp.dot(eb, w_ref[...], preferred_element_type=jnp.float32)
    o_ref[...] = pltpu.bitcast(tt.astype(jnp.bfloat16), jnp.int32)


def _edge_mm(e0, w3):
    return pl.pallas_call(
        _edge_mm_body,
        grid=(E // 2 // _BME,),
        in_specs=[_rows(_BME), _wspec()],
        out_specs=_rows(_BME),
        out_shape=jax.ShapeDtypeStruct((E // 2, D), jnp.int32),
    )(e0, w3)


def _edge_mm2_body(e0_ref, e1_ref, w_ref, o_ref):
    ea = pltpu.bitcast(e0_ref[...], jnp.bfloat16).astype(jnp.float32)
    eb = pltpu.bitcast(e1_ref[...], jnp.bfloat16).astype(jnp.float32)
    tt = jnp.dot((ea + eb).astype(jnp.bfloat16), w_ref[...],
                 preferred_element_type=jnp.float32)
    o_ref[...] = pltpu.bitcast(tt.astype(jnp.bfloat16), jnp.int32)


def _edge_mm2(e0, e1, w3):
    return pl.pallas_call(
        _edge_mm2_body,
        grid=(E // 2 // _BME,),
        in_specs=[_rows(_BME), _rows(_BME), _wspec()],
        out_specs=_rows(_BME),
        out_shape=jax.ShapeDtypeStruct((E // 2, D), jnp.int32),
    )(e0, e1, w3)


def _node_body(residual, prep, *refs):
    it = iter(refs)
    x_ref = next(it)
    aa_ref = next(it)
    ab_ref = next(it)
    wna_ref = next(it)
    wnb_ref = next(it)
    bn_ref = next(it)
    if prep:
        w1_ref = next(it)
        w2_ref = next(it)
        be_ref = next(it)
    h_ref = next(it)
    if prep:
        p1_ref = next(it)
        p2_ref = next(it)
    x = x_ref[...]
    agg = aa_ref[...] + ab_ref[...]
    h = jnp.maximum(
        jnp.dot(x, wna_ref[...], preferred_element_type=jnp.float32)
        + jnp.dot(agg, wnb_ref[...], preferred_element_type=jnp.float32)
        + bn_ref[...],
        0.0,
    )
    if residual:
        h = h + x
    h_ref[...] = h
    if prep:
        p1_ref[...] = jnp.dot(
            h, w1_ref[...], preferred_element_type=jnp.float32
        )
        p2_ref[...] = (
            jnp.dot(h, w2_ref[...], preferred_element_type=jnp.float32)
            + be_ref[...]
        )


def _node(residual, prep, x, agg_a, agg_b, wna, wnb, bn, *prep_args):
    n_out = 3 if prep else 1
    in_specs = [_rows(_BMN)] * 3 + [_wspec(), _wspec(), _bspec()]
    if prep:
        in_specs += [_wspec(), _wspec(), _bspec()]
    res = pl.pallas_call(
        functools.partial(_node_body, residual, prep),
        grid=(N // _BMN,),
        in_specs=in_specs,
        out_specs=[_rows(_BMN)] * n_out,
        out_shape=[jax.ShapeDtypeStruct((N, D), jnp.float32)] * n_out,
    )(x, agg_a, agg_b, wna, wnb, bn, *prep_args)
    return res if prep else res[0]


def kernel(edge_index, z, We0, be0, Wn0, bn0, We1, be1, Wn1, bn1,
           We2, be2, Wn2, bn2):
    nw = NC * NS
    src = edge_index[0]
    dst = edge_index[1]
    sd3 = jnp.transpose(
        edge_index[:, :EA].reshape(2, nw, CH, K), (1, 2, 0, 3)
    ).reshape(nw, 2 * CH, K)
    tidx = jnp.transpose(edge_index[:, EA:].reshape(2, nw, KT), (1, 0, 2))

    be0r = be0.reshape(1, D)
    be1r = be1.reshape(1, D)
    be2r = be2.reshape(1, D)
    bn0r = bn0.reshape(1, D)
    bn1r = bn1.reshape(1, D)
    bn2r = bn2.reshape(1, D)

    sc0 = _make_sc_kernel(has_t=False, write_e=True)
    sc1 = _make_sc_kernel(has_t=True, write_e=True)
    sc2 = _make_sc_kernel(has_t=True, write_e=False)

    p1, p2 = _prep0(z, We0[:D], We0[D:], be0r)
    e0, agg_a, agg_b = sc0(sd3, tidx, p1, p2)
    x1, p1, p2 = _node(False, True, z, agg_a, agg_b,
                       Wn0[:D], Wn0[D:], bn0r, We1[:D], We1[D:2 * D], be1r)

    t1 = _edge_mm(e0, We1[2 * D:].astype(jnp.bfloat16))
    e1, agg_a, agg_b = sc1(sd3, tidx, p1, p2, t1)
    x2, p1, p2 = _node(True, True, x1, agg_a, agg_b,
                       Wn1[:D], Wn1[D:], bn1r, We2[:D], We2[D:2 * D], be2r)

    t2 = _edge_mm2(e0, e1, We2[2 * D:].astype(jnp.bfloat16))
    agg_a, agg_b = sc2(sd3, tidx, p1, p2, t2)
    out = _node(False, False, x2, agg_a, agg_b, Wn2[:D], Wn2[D:], bn2r)
    return out


# speedup vs baseline: 1.0617x; 1.0010x over previous
import functools

import jax
import jax.numpy as jnp
from jax import lax
from jax.experimental import pallas as pl
from jax.experimental.pallas import tpu as pltpu
from jax.experimental.pallas import tpu_sc as plsc

N = 10000
E = 320000
D = 128

NC = 2
NS = 16
LN = 16

K = 64
CH = 156
KT = 16
EPW = CH * K + KT
EA = NC * NS * CH * K
IDXB = 16
NBLK = CH // IDXB
RPW = 632
RPW_LAST = N - 15 * RPW

_I32 = jnp.int32
_HI_MASK = -65536
_RND = 0x8000

_MESH = plsc.VectorSubcoreMesh(
    core_axis_name="c", subcore_axis_name="s", num_cores=NC, num_subcores=NS
)


def _sc_body(has_t, write_e, *refs):
    it = iter(refs)
    sd3 = next(it)
    tidx = next(it)
    p1 = next(it)
    p2 = next(it)
    t = next(it) if has_t else None
    e_out = next(it) if write_e else None
    agg_a = next(it)
    agg_b = next(it)
    idxs = next(it)
    idxt = next(it)
    abufs = (next(it), next(it))
    bbufs = (next(it), next(it))
    tbufs = (next(it), next(it))
    ebuf = next(it)
    agg_sh = next(it)
    sems_a = (next(it), next(it))
    sems_b = (next(it), next(it))
    sems_t = (next(it), next(it))

    c = lax.axis_index("c")
    s = lax.axis_index("s")
    wid = c * NS + s

    z0 = abufs[0]

    def _zrow(r, _):
        for j in range(D // LN):
            z0[r, pl.ds(j * LN, LN)] = jnp.zeros((LN,), jnp.float32)
        return 0

    lax.fori_loop(0, K, _zrow, 0)

    def _zero_span(base, rows):
        for q in range(rows // K):
            pltpu.sync_copy(z0, agg_sh.at[pl.ds(base + q * K, K)])
        rem = rows % K
        if rem:
            pltpu.sync_copy(z0.at[pl.ds(0, rem)],
                            agg_sh.at[pl.ds(base + rows - rem, rem)])

    @pl.when(s < NS - 1)
    def _():
        _zero_span(s * RPW, RPW)

    @pl.when(s == NS - 1)
    def _():
        _zero_span((NS - 1) * RPW, RPW_LAST)

    plsc.subcore_barrier()

    def _issue(cl, base_c, slot):
        prow = (wid * CH + base_c + cl) * (K // 2)
        pltpu.async_copy(p1.at[idxs.at[2 * cl]], abufs[slot], sems_a[slot])
        pltpu.async_copy(p2.at[idxs.at[2 * cl + 1]], bbufs[slot],
                         sems_b[slot])
        if has_t:
            pltpu.async_copy(t.at[pl.ds(prow, K // 2)], tbufs[slot],
                             sems_t[slot])

    def _wait(slot):
        dummy = p1.at[pl.ds(0, K)]
        pltpu.make_async_copy(dummy, abufs[slot], sems_a[slot]).wait()
        pltpu.make_async_copy(dummy, bbufs[slot], sems_b[slot]).wait()
        if has_t:
            dummy_t = t.at[pl.ds(0, K // 2)]
            pltpu.make_async_copy(dummy_t, tbufs[slot], sems_t[slot]).wait()

    def _pair_rows(ab, bb, tb, nrows):
        @plsc.parallel_loop(0, nrows, unroll=2)
        def _row(rr):
            r0 = 2 * rr
            r1 = r0 + 1
            for g in range(D // LN):
                sl = pl.ds(g * LN, LN)
                u = ab[r0, sl] + bb[r0, sl]
                v = ab[r1, sl] + bb[r1, sl]
                if has_t:
                    tw = tb[rr, sl]
                    u = u + lax.bitcast_convert_type(tw << 16, jnp.float32)
                    v = v + lax.bitcast_convert_type(tw & _HI_MASK,
                                                     jnp.float32)
                u = jnp.maximum(u, 0.0)
                v = jnp.maximum(v, 0.0)
                ab[r0, sl] = u
                ab[r1, sl] = v
                if write_e:
                    ub = lax.bitcast_convert_type(u, _I32)
                    vb = lax.bitcast_convert_type(v, _I32)
                    ebuf[rr, sl] = (
                        lax.shift_right_logical(ub + _RND, 16)
                        | ((vb + _RND) & _HI_MASK)
                    )

    def _run_block(base_c, bsz):
        pltpu.sync_copy(sd3.at[wid].at[pl.ds(2 * base_c, 2 * bsz)],
                        idxs.at[pl.ds(0, 2 * bsz)])
        _issue(0, base_c, 0)
        _issue(1, base_c, 1)

        def _pair(j2, _):
            for slot in (0, 1):
                cl = j2 * 2 + slot
                prow = (wid * CH + base_c + cl) * (K // 2)
                _wait(slot)
                _pair_rows(abufs[slot], bbufs[slot], tbufs[slot], K // 2)
                pltpu.sync_copy(abufs[slot], agg_sh.at[idxs.at[2 * cl + 1]],
                                add=True)
                if write_e:
                    pltpu.sync_copy(ebuf, e_out.at[pl.ds(prow, K // 2)])

                @pl.when(cl + 2 < bsz)
                def _(cl=cl, slot=slot):
                    _issue(cl + 2, base_c, slot)
            return 0

        lax.fori_loop(0, bsz // 2, _pair, 0)

    def _block(blk, _):
        _run_block(pl.multiple_of(blk * IDXB, IDXB), IDXB)
        return 0

    lax.fori_loop(0, NBLK, _block, 0)
    if CH % IDXB:
        _run_block(NBLK * IDXB, CH % IDXB)

    pltpu.sync_copy(tidx.at[wid], idxt)
    tprow = EA // 2 + wid * (KT // 2)
    cpa = pltpu.async_copy(p1.at[idxt.at[0]], abufs[0].at[pl.ds(0, KT)],
                           sems_a[0])
    cpb = pltpu.async_copy(p2.at[idxt.at[1]], bbufs[0].at[pl.ds(0, KT)],
                           sems_b[0])
    if has_t:
        cpt = pltpu.async_copy(t.at[pl.ds(tprow, KT // 2)],
                               tbufs[0].at[pl.ds(0, KT // 2)], sems_t[0])
    cpa.wait()
    cpb.wait()
    if has_t:
        cpt.wait()
    _pair_rows(abufs[0], bbufs[0], tbufs[0], KT // 2)
    pltpu.sync_copy(abufs[0].at[pl.ds(0, KT)], agg_sh.at[idxt.at[1]],
                    add=True)
    if write_e:
        pltpu.sync_copy(ebuf.at[pl.ds(0, KT // 2)],
                        e_out.at[pl.ds(tprow, KT // 2)])

    plsc.subcore_barrier()

    def _flush(agg_out):
        @pl.when(s < NS - 1)
        def _():
            pltpu.sync_copy(agg_sh.at[pl.ds(s * RPW, RPW)],
                            agg_out.at[pl.ds(s * RPW, RPW)])

        @pl.when(s == NS - 1)
        def _():
            pltpu.sync_copy(agg_sh.at[pl.ds((NS - 1) * RPW, RPW_LAST)],
                            agg_out.at[pl.ds((NS - 1) * RPW, RPW_LAST)])

    @pl.when(c == 0)
    def _():
        _flush(agg_a)

    @pl.when(c == 1)
    def _():
        _flush(agg_b)


def _make_sc_kernel(has_t, write_e):
    f32 = jnp.float32
    outs = []
    if write_e:
        outs.append(jax.ShapeDtypeStruct((E // 2, D), jnp.int32))
    outs.append(jax.ShapeDtypeStruct((N, D), f32))
    outs.append(jax.ShapeDtypeStruct((N, D), f32))
    scratch = (
        [pltpu.VMEM((2 * IDXB, K), jnp.int32)]
        + [pltpu.VMEM((2, KT), jnp.int32)]
        + [pltpu.VMEM((K, D), f32)] * 4
        + [pltpu.VMEM((K // 2, D), jnp.int32)] * 2
        + [pltpu.VMEM((K // 2, D), jnp.int32)]
        + [pltpu.VMEM_SHARED((N, D), f32)]
        + [pltpu.SemaphoreType.DMA] * 6
    )
    return pl.kernel(
        functools.partial(_sc_body, has_t, write_e),
        out_type=tuple(outs),
        mesh=_MESH,
        scratch_types=scratch,
    )


_BMN = 2000
_BME = 16000


def _wspec():
    return pl.BlockSpec((D, D), lambda i: (0, 0))


def _bspec():
    return pl.BlockSpec((1, D), lambda i: (0, 0))


def _rows(bm):
    return pl.BlockSpec((bm, D), lambda i: (i, 0))


def _prep0_body(x_ref, w1_ref, w2_ref, be_ref, p1_ref, p2_ref):
    x = x_ref[...]
    p1_ref[...] = jnp.dot(x, w1_ref[...], preferred_element_type=jnp.float32)
    p2_ref[...] = (
        jnp.dot(x, w2_ref[...], preferred_element_type=jnp.float32)
        + be_ref[...]
    )


def _prep0(z, w1, w2, be):
    return pl.pallas_call(
        _prep0_body,
        grid=(N // _BMN,),
        in_specs=[_rows(_BMN), _wspec(), _wspec(), _bspec()],
        out_specs=[_rows(_BMN), _rows(_BMN)],
        out_shape=[jax.ShapeDtypeStruct((N, D), jnp.float32)] * 2,
    )(z, w1, w2, be)


def _edge_mm_body(e_ref, w_ref, o_ref):
    eb = pltpu.bitcast(e_ref[...], jnp.bfloat16)
    tt = jnp.dot(eb, w_ref[...], preferred_element_type=jnp.float32)
    o_ref[...] = pltpu.bitcast(tt.astype(jnp.bfloat16), jnp.int32)


def _edge_mm(e0, w3):
    return pl.pallas_call(
        _edge_mm_body,
        grid=(E // 2 // _BME,),
        in_specs=[_rows(_BME), _wspec()],
        out_specs=_rows(_BME),
        out_shape=jax.ShapeDtypeStruct((E // 2, D), jnp.int32),
    )(e0, w3)


def _edge_mm2_body(e0_ref, e1_ref, w_ref, o_ref):
    ea = pltpu.bitcast(e0_ref[...], jnp.bfloat16).astype(jnp.float32)
    eb = pltpu.bitcast(e1_ref[...], jnp.bfloat16).astype(jnp.float32)
    tt = jnp.dot((ea + eb).astype(jnp.bfloat16), w_ref[...],
                 preferred_element_type=jnp.float32)
    o_ref[...] = pltpu.bitcast(tt.astype(jnp.bfloat16), jnp.int32)


def _edge_mm2(e0, e1, w3):
    return pl.pallas_call(
        _edge_mm2_body,
        grid=(E // 2 // _BME,),
        in_specs=[_rows(_BME), _rows(_BME), _wspec()],
        out_specs=_rows(_BME),
        out_shape=jax.ShapeDtypeStruct((E // 2, D), jnp.int32),
    )(e0, e1, w3)


def _node_body(residual, prep, *refs):
    it = iter(refs)
    x_ref = next(it)
    aa_ref = next(it)
    ab_ref = next(it)
    wna_ref = next(it)
    wnb_ref = next(it)
    bn_ref = next(it)
    if prep:
        w1_ref = next(it)
        w2_ref = next(it)
        be_ref = next(it)
    h_ref = next(it)
    if prep:
        p1_ref = next(it)
        p2_ref = next(it)
    x = x_ref[...]
    agg = aa_ref[...] + ab_ref[...]
    h = jnp.maximum(
        jnp.dot(x, wna_ref[...], preferred_element_type=jnp.float32)
        + jnp.dot(agg, wnb_ref[...], preferred_element_type=jnp.float32)
        + bn_ref[...],
        0.0,
    )
    if residual:
        h = h + x
    h_ref[...] = h
    if prep:
        p1_ref[...] = jnp.dot(
            h, w1_ref[...], preferred_element_type=jnp.float32
        )
        p2_ref[...] = (
            jnp.dot(h, w2_ref[...], preferred_element_type=jnp.float32)
            + be_ref[...]
        )


def _node(residual, prep, x, agg_a, agg_b, wna, wnb, bn, *prep_args):
    n_out = 3 if prep else 1
    in_specs = [_rows(_BMN)] * 3 + [_wspec(), _wspec(), _bspec()]
    if prep:
        in_specs += [_wspec(), _wspec(), _bspec()]
    res = pl.pallas_call(
        functools.partial(_node_body, residual, prep),
        grid=(N // _BMN,),
        in_specs=in_specs,
        out_specs=[_rows(_BMN)] * n_out,
        out_shape=[jax.ShapeDtypeStruct((N, D), jnp.float32)] * n_out,
    )(x, agg_a, agg_b, wna, wnb, bn, *prep_args)
    return res if prep else res[0]


def kernel(edge_index, z, We0, be0, Wn0, bn0, We1, be1, Wn1, bn1,
           We2, be2, Wn2, bn2):
    nw = NC * NS
    src = edge_index[0]
    dst = edge_index[1]
    sd3 = jnp.transpose(
        edge_index[:, :EA].reshape(2, nw, CH, K), (1, 2, 0, 3)
    ).reshape(nw, 2 * CH, K)
    tidx = jnp.transpose(edge_index[:, EA:].reshape(2, nw, KT), (1, 0, 2))

    be0r = be0.reshape(1, D)
    be1r = be1.reshape(1, D)
    be2r = be2.reshape(1, D)
    bn0r = bn0.reshape(1, D)
    bn1r = bn1.reshape(1, D)
    bn2r = bn2.reshape(1, D)

    sc0 = _make_sc_kernel(has_t=False, write_e=True)
    sc1 = _make_sc_kernel(has_t=True, write_e=True)
    sc2 = _make_sc_kernel(has_t=True, write_e=False)

    p1, p2 = _prep0(z, We0[:D], We0[D:], be0r)
    e0, agg_a, agg_b = sc0(sd3, tidx, p1, p2)
    x1, p1, p2 = _node(False, True, z, agg_a, agg_b,
                       Wn0[:D], Wn0[D:], bn0r, We1[:D], We1[D:2 * D], be1r)

    t1 = _edge_mm(e0, We1[2 * D:].astype(jnp.bfloat16))
    e1, agg_a, agg_b = sc1(sd3, tidx, p1, p2, t1)
    x2, p1, p2 = _node(True, True, x1, agg_a, agg_b,
                       Wn1[:D], Wn1[D:], bn1r, We2[:D], We2[D:2 * D], be2r)

    t2 = _edge_mm2(e0, e1, We2[2 * D:].astype(jnp.bfloat16))
    agg_a, agg_b = sc2(sd3, tidx, p1, p2, t2)
    out = _node(False, False, x2, agg_a, agg_b, Wn2[:D], Wn2[D:], bn2r)
    return out
